# Initial kernel scaffold; baseline (speedup 1.0000x reference)
import jax
import jax.numpy as jnp
from jax.experimental import pallas as pl


def kernel(x, edge_index, W0, b0, W1, b1, W2, b2, W3, b3, W4, b4, W5, b5, gw1, gb1, gw2, gb2, m1, mb1, m2, mb2, m3, mb3, m4, mb4):
    raise NotImplementedError("write your pallas kernel here")


# speedup vs baseline: 28.9000x; 28.9000x over previous
import functools

import jax
import jax.numpy as jnp
from jax import lax
from jax.experimental import pallas as pl
from jax.experimental.pallas import tpu as pltpu
from jax.experimental.pallas import tpu_sc as plsc

N = 10000
E = 320000
F = 128
D = 64

NC = 2
NS = 16
NW = NC * NS
RPT = N // NS
CH = 125
KC = E // (NW * CH)
NBUF = 4
LAG = 2

_mesh = plsc.VectorSubcoreMesh(core_axis_name="c", subcore_axis_name="s")


@functools.partial(
    pl.kernel,
    out_type=jax.ShapeDtypeStruct((NW, RPT, 16), jnp.float32),
    mesh=_mesh,
    compiler_params=pltpu.CompilerParams(use_tc_tiling_on_sc=False),
    scratch_types=[
        pltpu.VMEM((KC, CH), jnp.int32),
        pltpu.VMEM((CH, 16), jnp.float32),
        pltpu.VMEM_SHARED((N, 16), jnp.float32),
        pltpu.SemaphoreType.DMA,
    ],
)
def _sc_degree(dst_hbm, zeros16_hbm, ones_hbm, out_hbm, didx, ones, acc, ssem):
  c = lax.axis_index("c")
  s = lax.axis_index("s")
  w = c * NS + s
  pltpu.sync_copy(dst_hbm.at[pl.ds(w * KC, KC)], didx)
  pltpu.sync_copy(ones_hbm, ones)
  pltpu.sync_copy(zeros16_hbm, acc.at[pl.ds(s * RPT, RPT)])
  plsc.subcore_barrier()

  for p in range(NBUF):
    pltpu.async_copy(ones, acc.at[didx.at[p]], ssem, add=True)

  @pl.loop(NBUF, KC)
  def _(i):
    pltpu.make_async_copy(ones, acc.at[didx.at[i - NBUF]], ssem).wait()
    pltpu.async_copy(ones, acc.at[didx.at[i]], ssem, add=True)

  for p in range(NBUF):
    pltpu.make_async_copy(ones, acc.at[didx.at[KC - NBUF + p]], ssem).wait()

  plsc.subcore_barrier()
  pltpu.sync_copy(acc.at[pl.ds(s * RPT, RPT)], out_hbm.at[w])


@functools.partial(
    pl.kernel,
    out_type=jax.ShapeDtypeStruct((NW, RPT, D), jnp.float32),
    mesh=_mesh,
    compiler_params=pltpu.CompilerParams(use_tc_tiling_on_sc=False),
    scratch_types=[
        pltpu.VMEM((KC, CH), jnp.int32),
        pltpu.VMEM((KC, CH), jnp.int32),
        [pltpu.VMEM((CH, D), jnp.float32) for _ in range(NBUF)],
        pltpu.VMEM_SHARED((N, D), jnp.float32),
        [pltpu.SemaphoreType.DMA for _ in range(NBUF)],
        [pltpu.SemaphoreType.DMA for _ in range(NBUF)],
    ],
)
def _sc_edge_sum(hws_hbm, src_hbm, dst_hbm, zeros_hbm, out_hbm,
                 sidx, didx, rows, acc, gsem, ssem):
  c = lax.axis_index("c")
  s = lax.axis_index("s")
  w = c * NS + s
  pltpu.sync_copy(src_hbm.at[pl.ds(w * KC, KC)], sidx)
  pltpu.sync_copy(dst_hbm.at[pl.ds(w * KC, KC)], didx)
  pltpu.sync_copy(zeros_hbm, acc.at[pl.ds(s * RPT, RPT)])
  plsc.subcore_barrier()

  @pl.loop(0, KC + NBUF, step=NBUF)
  def _(i):
    for p in range(NBUF):
      cg = i + p
      b = p
      bs = (p - LAG) % NBUF

      @pl.when(cg >= NBUF)
      def _():
        pltpu.make_async_copy(rows[b], acc.at[didx.at[cg - NBUF]],
                              ssem[b]).wait()

      @pl.when(cg < KC)
      def _():
        pltpu.async_copy(hws_hbm.at[sidx.at[cg]], rows[b], gsem[b])

      cs = cg - LAG

      @pl.when(jnp.logical_and(cs >= 0, cs < KC))
      def _():
        pltpu.make_async_copy(hws_hbm.at[sidx.at[cs]], rows[bs],
                              gsem[bs]).wait()
        pltpu.async_copy(rows[bs], acc.at[didx.at[cs]], ssem[bs], add=True)

  plsc.subcore_barrier()
  pltpu.sync_copy(acc.at[pl.ds(s * RPT, RPT)], out_hbm.at[w])


BM = 400
GRID = N // BM


def _t1_body(x_ref, w0_ref, d0_ref, d1_ref, dinv_ref, hws_ref):
  deg = d0_ref[...] + d1_ref[...] + 1.0
  dinv = lax.rsqrt(deg)
  dinv_ref[...] = dinv
  hw = jnp.dot(x_ref[...], w0_ref[...], preferred_element_type=jnp.float32)
  hws_ref[...] = dinv[:, :1] * hw


_t1 = pl.pallas_call(
    _t1_body,
    grid=(GRID,),
    in_specs=[
        pl.BlockSpec((BM, F), lambda i: (i, 0)),
        pl.BlockSpec((F, D), lambda i: (0, 0)),
        pl.BlockSpec((BM, 16), lambda i: (i, 0)),
        pl.BlockSpec((BM, 16), lambda i: (i + GRID, 0)),
    ],
    out_specs=[
        pl.BlockSpec((BM, 16), lambda i: (i, 0)),
        pl.BlockSpec((BM, D), lambda i: (i, 0)),
    ],
    out_shape=[
        jax.ShapeDtypeStruct((N, 16), jnp.float32),
        jax.ShapeDtypeStruct((N, D), jnp.float32),
    ],
)


def _t2_body(s0_ref, s1_ref, hws_ref, dinv_ref, m_ref, b_ref, w_ref,
             hws_out_ref, m_out_ref):
  dinv = dinv_ref[...][:, :1]
  h = jnp.maximum(
      dinv * (s0_ref[...] + s1_ref[...] + hws_ref[...]) + b_ref[...], 0.0)
  m_out_ref[...] = jnp.maximum(m_ref[...], h)
  hw = jnp.dot(h, w_ref[...], preferred_element_type=jnp.float32)
  hws_out_ref[...] = dinv * hw


_t2 = pl.pallas_call(
    _t2_body,
    grid=(GRID,),
    in_specs=[
        pl.BlockSpec((BM, D), lambda i: (i, 0)),
        pl.BlockSpec((BM, D), lambda i: (i + GRID, 0)),
        pl.BlockSpec((BM, D), lambda i: (i, 0)),
        pl.BlockSpec((BM, 16), lambda i: (i, 0)),
        pl.BlockSpec((BM, D), lambda i: (i, 0)),
        pl.BlockSpec((1, D), lambda i: (0, 0)),
        pl.BlockSpec((D, D), lambda i: (0, 0)),
    ],
    out_specs=[
        pl.BlockSpec((BM, D), lambda i: (i, 0)),
        pl.BlockSpec((BM, D), lambda i: (i, 0)),
    ],
    out_shape=[
        jax.ShapeDtypeStruct((N, D), jnp.float32),
        jax.ShapeDtypeStruct((N, D), jnp.float32),
    ],
)


def _t3_body(s0_ref, s1_ref, hws_ref, dinv_ref, m_ref, b_ref,
             gw1_ref, gb1_ref, gw2_ref, gb2_ref,
             m1_ref, mb1_ref, m2_ref, mb2_ref, m3_ref, mb3_ref,
             m4_ref, mb4_ref, out_ref):
  dinv = dinv_ref[...][:, :1]
  h = jnp.maximum(
      dinv * (s0_ref[...] + s1_ref[...] + hws_ref[...]) + b_ref[...], 0.0)
  m = jnp.maximum(m_ref[...], h)
  g = jnp.maximum(
      jnp.dot(m, gw1_ref[...], preferred_element_type=jnp.float32)
      + gb1_ref[...], 0.0)
  gate = jnp.dot(g, gw2_ref[...], preferred_element_type=jnp.float32) \
      + gb2_ref[...]
  gate = gate - jnp.max(gate)
  e = jnp.exp(gate)
  alpha = e / jnp.sum(e)
  pooled = jnp.sum(alpha * m, axis=0, keepdims=True)
  z = jnp.maximum(
      jnp.dot(pooled, m1_ref[...], preferred_element_type=jnp.float32)
      + mb1_ref[...], 0.0)
  z = jnp.maximum(
      jnp.dot(z, m2_ref[...], preferred_element_type=jnp.float32)
      + mb2_ref[...], 0.0)
  z = jnp.maximum(
      jnp.dot(z, m3_ref[...], preferred_element_type=jnp.float32)
      + mb3_ref[...], 0.0)
  out_ref[...] = jnp.dot(z, m4_ref[...], preferred_element_type=jnp.float32) \
      + mb4_ref[...]


def _t3(sparts, hws, dinv16, m, b5, gw1, gb1, gw2, gb2,
        m1, mb1, m2, mb2, m3, mb3, m4, mb4):
  return pl.pallas_call(
      _t3_body,
      grid=(1,),
      in_specs=[
          pl.BlockSpec((N, D), lambda i: (0, 0)),
          pl.BlockSpec((N, D), lambda i: (1, 0)),
          pl.BlockSpec((N, D), lambda i: (0, 0)),
          pl.BlockSpec((N, 16), lambda i: (0, 0)),
          pl.BlockSpec((N, D), lambda i: (0, 0)),
          pl.BlockSpec((1, D), lambda i: (0, 0)),
          pl.BlockSpec((D, D), lambda i: (0, 0)),
          pl.BlockSpec((1, D), lambda i: (0, 0)),
          pl.BlockSpec((D, 1), lambda i: (0, 0)),
          pl.BlockSpec((1, 1), lambda i: (0, 0)),
          pl.BlockSpec((D, D // 2), lambda i: (0, 0)),
          pl.BlockSpec((1, D // 2), lambda i: (0, 0)),
          pl.BlockSpec((D // 2, D // 4), lambda i: (0, 0)),
          pl.BlockSpec((1, D // 4), lambda i: (0, 0)),
          pl.BlockSpec((D // 4, D // 8), lambda i: (0, 0)),
          pl.BlockSpec((1, D // 8), lambda i: (0, 0)),
          pl.BlockSpec((D // 8, 1), lambda i: (0, 0)),
          pl.BlockSpec((1, 1), lambda i: (0, 0)),
      ],
      out_specs=pl.BlockSpec((1, 1), lambda i: (0, 0)),
      out_shape=jax.ShapeDtypeStruct((1, 1), jnp.float32),
  )(sparts, sparts, hws, dinv16, m, b5, gw1, gb1, gw2, gb2,
    m1, mb1, m2, mb2, m3, mb3, m4, mb4)


def kernel(x, edge_index, W0, b0, W1, b1, W2, b2, W3, b3, W4, b4, W5, b5,
           gw1, gb1, gw2, gb2, m1, mb1, m2, mb2, m3, mb3, m4, mb4):
  src2 = edge_index[0].reshape(NW * KC, CH)
  dst2 = edge_index[1].reshape(NW * KC, CH)
  zeros = jnp.zeros((RPT, D), jnp.float32)
  zeros16 = jnp.zeros((RPT, 16), jnp.float32)
  ones16 = jnp.ones((CH, 16), jnp.float32)

  dparts = _sc_degree(dst2, zeros16, ones16).reshape(NC * N, 16)
  dinv16, hws = _t1(x, W0, dparts, dparts)

  bs = [b0.reshape(1, D), b1.reshape(1, D), b2.reshape(1, D),
        b3.reshape(1, D), b4.reshape(1, D), b5.reshape(1, D)]
  Ws = [W1, W2, W3, W4, W5]
  m = jnp.zeros((N, D), jnp.float32)
  for l in range(5):
    sparts = _sc_edge_sum(hws, src2, dst2, zeros).reshape(NC * N, D)
    hws, m = _t2(sparts, sparts, hws, dinv16, m, bs[l], Ws[l])
  sparts = _sc_edge_sum(hws, src2, dst2, zeros).reshape(NC * N, D)
  out = _t3(sparts, hws, dinv16, m, bs[5], gw1, gb1.reshape(1, D),
            gw2, gb2.reshape(1, 1),
            m1, mb1.reshape(1, D // 2), m2, mb2.reshape(1, D // 4),
            m3, mb3.reshape(1, D // 8), m4, mb4.reshape(1, 1))
  return out.reshape(1)


# speedup vs baseline: 32.8525x; 1.1368x over previous
import functools

import jax
import jax.numpy as jnp
from jax import lax
from jax.experimental import pallas as pl
from jax.experimental.pallas import tpu as pltpu
from jax.experimental.pallas import tpu_sc as plsc

N = 10000
E = 320000
F = 128
D = 64

NC = 2
NS = 16
NW = NC * NS
RPT = N // NS
CH = 125
KC = E // (NW * CH)
NBUF = 8
LAG = 4

_mesh = plsc.VectorSubcoreMesh(core_axis_name="c", subcore_axis_name="s")


@functools.partial(
    pl.kernel,
    out_type=jax.ShapeDtypeStruct((NC * N, 16), jnp.float32),
    mesh=_mesh,
    compiler_params=pltpu.CompilerParams(use_tc_tiling_on_sc=False),
    scratch_types=[
        pltpu.VMEM((KC, CH), jnp.int32),
        pltpu.VMEM((CH, 16), jnp.float32),
        pltpu.VMEM_SHARED((N, 16), jnp.float32),
        pltpu.SemaphoreType.DMA,
    ],
)
def _sc_degree(dst_hbm, zeros16_hbm, ones_hbm, out_hbm, didx, ones, acc, ssem):
  c = lax.axis_index("c")
  s = lax.axis_index("s")
  w = c * NS + s
  pltpu.sync_copy(dst_hbm.at[pl.ds(w * KC, KC)], didx)
  pltpu.sync_copy(ones_hbm, ones)
  pltpu.sync_copy(zeros16_hbm, acc.at[pl.ds(s * RPT, RPT)])
  plsc.subcore_barrier()

  for p in range(NBUF):
    pltpu.async_copy(ones, acc.at[didx.at[p]], ssem, add=True)

  @pl.loop(NBUF, KC)
  def _(i):
    pltpu.make_async_copy(ones, acc.at[didx.at[i - NBUF]], ssem).wait()
    pltpu.async_copy(ones, acc.at[didx.at[i]], ssem, add=True)

  for p in range(NBUF):
    pltpu.make_async_copy(ones, acc.at[didx.at[KC - NBUF + p]], ssem).wait()

  plsc.subcore_barrier()
  pltpu.sync_copy(acc.at[pl.ds(s * RPT, RPT)],
                  out_hbm.at[pl.ds(c * N + s * RPT, RPT)])


@functools.partial(
    pl.kernel,
    out_type=jax.ShapeDtypeStruct((NC * N, D), jnp.float32),
    mesh=_mesh,
    compiler_params=pltpu.CompilerParams(use_tc_tiling_on_sc=False),
    scratch_types=[
        pltpu.VMEM((KC, CH), jnp.int32),
        pltpu.VMEM((KC, CH), jnp.int32),
        [pltpu.VMEM((CH, D), jnp.float32) for _ in range(NBUF)],
        pltpu.VMEM_SHARED((N, D), jnp.float32),
        [pltpu.SemaphoreType.DMA for _ in range(NBUF)],
        [pltpu.SemaphoreType.DMA for _ in range(NBUF)],
    ],
)
def _sc_edge_sum(hws_hbm, src_hbm, dst_hbm, zeros_hbm, out_hbm,
                 sidx, didx, rows, acc, gsem, ssem):
  c = lax.axis_index("c")
  s = lax.axis_index("s")
  w = c * NS + s
  pltpu.sync_copy(src_hbm.at[pl.ds(w * KC, KC)], sidx)
  pltpu.sync_copy(dst_hbm.at[pl.ds(w * KC, KC)], didx)
  pltpu.sync_copy(zeros_hbm, acc.at[pl.ds(s * RPT, RPT)])
  plsc.subcore_barrier()

  @pl.loop(0, KC + NBUF, step=NBUF)
  def _(i):
    for p in range(NBUF):
      cg = i + p
      b = p
      bs = (p - LAG) % NBUF

      @pl.when(cg >= NBUF)
      def _():
        pltpu.make_async_copy(rows[b], acc.at[didx.at[cg - NBUF]],
                              ssem[b]).wait()

      @pl.when(cg < KC)
      def _():
        pltpu.async_copy(hws_hbm.at[sidx.at[cg]], rows[b], gsem[b])

      cs = cg - LAG

      @pl.when(jnp.logical_and(cs >= 0, cs < KC))
      def _():
        pltpu.make_async_copy(hws_hbm.at[sidx.at[cs]], rows[bs],
                              gsem[bs]).wait()
        pltpu.async_copy(rows[bs], acc.at[didx.at[cs]], ssem[bs], add=True)

  plsc.subcore_barrier()
  pltpu.sync_copy(acc.at[pl.ds(s * RPT, RPT)],
                  out_hbm.at[pl.ds(c * N + s * RPT, RPT)])


BM = 2000
GRID = N // BM


def _t1a_body(x_ref, w0_ref, hw_ref):
  hw_ref[...] = jnp.dot(x_ref[...], w0_ref[...],
                        preferred_element_type=jnp.float32)


_t1a = pl.pallas_call(
    _t1a_body,
    grid=(GRID,),
    in_specs=[
        pl.BlockSpec((BM, F), lambda i: (i, 0)),
        pl.BlockSpec((F, D), lambda i: (0, 0)),
    ],
    out_specs=pl.BlockSpec((BM, D), lambda i: (i, 0)),
    out_shape=jax.ShapeDtypeStruct((N, D), jnp.float32),
)


def _t1b_body(hw_ref, d0_ref, d1_ref, dinv_ref, hws_ref):
  deg = d0_ref[...] + d1_ref[...] + 1.0
  dinv = lax.rsqrt(deg)
  dinv_ref[...] = dinv
  hws_ref[...] = dinv[:, :1] * hw_ref[...]


_t1b = pl.pallas_call(
    _t1b_body,
    grid=(GRID,),
    in_specs=[
        pl.BlockSpec((BM, D), lambda i: (i, 0)),
        pl.BlockSpec((BM, 16), lambda i: (i, 0)),
        pl.BlockSpec((BM, 16), lambda i: (i + GRID, 0)),
    ],
    out_specs=[
        pl.BlockSpec((BM, 16), lambda i: (i, 0)),
        pl.BlockSpec((BM, D), lambda i: (i, 0)),
    ],
    out_shape=[
        jax.ShapeDtypeStruct((N, 16), jnp.float32),
        jax.ShapeDtypeStruct((N, D), jnp.float32),
    ],
)


def _t2_body(s0_ref, s1_ref, hws_ref, dinv_ref, m_ref, b_ref, w_ref,
             hws_out_ref, m_out_ref):
  dinv = dinv_ref[...][:, :1]
  h = jnp.maximum(
      dinv * (s0_ref[...] + s1_ref[...] + hws_ref[...]) + b_ref[...], 0.0)
  m_out_ref[...] = jnp.maximum(m_ref[...], h)
  hw = jnp.dot(h, w_ref[...], preferred_element_type=jnp.float32)
  hws_out_ref[...] = dinv * hw


_t2 = pl.pallas_call(
    _t2_body,
    grid=(GRID,),
    in_specs=[
        pl.BlockSpec((BM, D), lambda i: (i, 0)),
        pl.BlockSpec((BM, D), lambda i: (i + GRID, 0)),
        pl.BlockSpec((BM, D), lambda i: (i, 0)),
        pl.BlockSpec((BM, 16), lambda i: (i, 0)),
        pl.BlockSpec((BM, D), lambda i: (i, 0)),
        pl.BlockSpec((1, D), lambda i: (0, 0)),
        pl.BlockSpec((D, D), lambda i: (0, 0)),
    ],
    out_specs=[
        pl.BlockSpec((BM, D), lambda i: (i, 0)),
        pl.BlockSpec((BM, D), lambda i: (i, 0)),
    ],
    out_shape=[
        jax.ShapeDtypeStruct((N, D), jnp.float32),
        jax.ShapeDtypeStruct((N, D), jnp.float32),
    ],
)


def _t3_body(s0_ref, s1_ref, hws_ref, dinv_ref, m_ref, b_ref,
             gw1_ref, gb1_ref, gw2_ref, gb2_ref,
             m1_ref, mb1_ref, m2_ref, mb2_ref, m3_ref, mb3_ref,
             m4_ref, mb4_ref, out_ref):
  dinv = dinv_ref[...][:, :1]
  h = jnp.maximum(
      dinv * (s0_ref[...] + s1_ref[...] + hws_ref[...]) + b_ref[...], 0.0)
  m = jnp.maximum(m_ref[...], h)
  g = jnp.maximum(
      jnp.dot(m, gw1_ref[...], preferred_element_type=jnp.float32)
      + gb1_ref[...], 0.0)
  gate = jnp.dot(g, gw2_ref[...], preferred_element_type=jnp.float32) \
      + gb2_ref[...]
  gate = gate - jnp.max(gate)
  e = jnp.exp(gate)
  alpha = e / jnp.sum(e)
  pooled = jnp.sum(alpha * m, axis=0, keepdims=True)
  z = jnp.maximum(
      jnp.dot(pooled, m1_ref[...], preferred_element_type=jnp.float32)
      + mb1_ref[...], 0.0)
  z = jnp.maximum(
      jnp.dot(z, m2_ref[...], preferred_element_type=jnp.float32)
      + mb2_ref[...], 0.0)
  z = jnp.maximum(
      jnp.dot(z, m3_ref[...], preferred_element_type=jnp.float32)
      + mb3_ref[...], 0.0)
  out_ref[...] = jnp.dot(z, m4_ref[...], preferred_element_type=jnp.float32) \
      + mb4_ref[...]


def _t3(sparts, hws, dinv16, m, b5, gw1, gb1, gw2, gb2,
        m1, mb1, m2, mb2, m3, mb3, m4, mb4):
  return pl.pallas_call(
      _t3_body,
      grid=(1,),
      in_specs=[
          pl.BlockSpec((N, D), lambda i: (0, 0)),
          pl.BlockSpec((N, D), lambda i: (1, 0)),
          pl.BlockSpec((N, D), lambda i: (0, 0)),
          pl.BlockSpec((N, 16), lambda i: (0, 0)),
          pl.BlockSpec((N, D), lambda i: (0, 0)),
          pl.BlockSpec((1, D), lambda i: (0, 0)),
          pl.BlockSpec((D, D), lambda i: (0, 0)),
          pl.BlockSpec((1, D), lambda i: (0, 0)),
          pl.BlockSpec((D, 1), lambda i: (0, 0)),
          pl.BlockSpec((1, 1), lambda i: (0, 0)),
          pl.BlockSpec((D, D // 2), lambda i: (0, 0)),
          pl.BlockSpec((1, D // 2), lambda i: (0, 0)),
          pl.BlockSpec((D // 2, D // 4), lambda i: (0, 0)),
          pl.BlockSpec((1, D // 4), lambda i: (0, 0)),
          pl.BlockSpec((D // 4, D // 8), lambda i: (0, 0)),
          pl.BlockSpec((1, D // 8), lambda i: (0, 0)),
          pl.BlockSpec((D // 8, 1), lambda i: (0, 0)),
          pl.BlockSpec((1, 1), lambda i: (0, 0)),
      ],
      out_specs=pl.BlockSpec((1, 1), lambda i: (0, 0)),
      out_shape=jax.ShapeDtypeStruct((1, 1), jnp.float32),
  )(sparts, sparts, hws, dinv16, m, b5, gw1, gb1, gw2, gb2,
    m1, mb1, m2, mb2, m3, mb3, m4, mb4)


def kernel(x, edge_index, W0, b0, W1, b1, W2, b2, W3, b3, W4, b4, W5, b5,
           gw1, gb1, gw2, gb2, m1, mb1, m2, mb2, m3, mb3, m4, mb4):
  src2 = edge_index[0].reshape(NW * KC, CH)
  dst2 = edge_index[1].reshape(NW * KC, CH)
  zeros = jnp.zeros((RPT, D), jnp.float32)
  zeros16 = jnp.zeros((RPT, 16), jnp.float32)
  ones16 = jnp.ones((CH, 16), jnp.float32)

  hw0 = _t1a(x, W0)
  dparts = _sc_degree(dst2, zeros16, ones16)
  dinv16, hws = _t1b(hw0, dparts, dparts)

  bs = [b0.reshape(1, D), b1.reshape(1, D), b2.reshape(1, D),
        b3.reshape(1, D), b4.reshape(1, D), b5.reshape(1, D)]
  Ws = [W1, W2, W3, W4, W5]
  m = jnp.zeros((N, D), jnp.float32)
  for l in range(5):
    sparts = _sc_edge_sum(hws, src2, dst2, zeros)
    hws, m = _t2(sparts, sparts, hws, dinv16, m, bs[l], Ws[l])
  sparts = _sc_edge_sum(hws, src2, dst2, zeros)
  out = _t3(sparts, hws, dinv16, m, bs[5], gw1, gb1.reshape(1, D),
            gw2, gb2.reshape(1, 1),
            m1, mb1.reshape(1, D // 2), m2, mb2.reshape(1, D // 4),
            m3, mb3.reshape(1, D // 8), m4, mb4.reshape(1, 1))
  return out.reshape(1)
